# baseline (speedup 1.0000x reference)
import functools

import jax
import jax.numpy as jnp
from jax import lax
from jax.experimental import pallas as pl
from jax.experimental.pallas import tpu as pltpu
from jax.experimental.pallas import tpu_sc as plsc

N = 10000
E = 320000
NP = 10240
EP = 327680
CH = 128
NCH = EP // CH
NTILES = 16
CPT = NCH // NTILES
RPT = NP // NTILES

SROWS = 2560
ROWS = 2000

BLK = 8
NBLK = CPT // BLK
GS = (3, 3, 2)

DBLK = 16
DNBLK = CPT // DBLK


def _rs(d):
    return lax.rsqrt(jnp.where(d > 0, d, 1.0))


def _b1_body(x_ref, w_ref, b_ref, dego_ref, sa_ref, sb_ref):
    s = jnp.dot(x_ref[...], w_ref[...], preferred_element_type=jnp.float32)
    s = (s + b_ref[...]) * _rs(dego_ref[...])
    sa_ref[...] = s[:, :128]
    sb_ref[...] = s[:, 128:]


def _b2_body(oa_ref, ob_ref, degi_ref, dego_ref, w_ref, b_ref, sa_ref, sb_ref):
    o = jnp.concatenate([oa_ref[...], ob_ref[...]], axis=1)
    h = jax.nn.relu(o * _rs(degi_ref[...]))
    s = jnp.dot(h, w_ref[...], preferred_element_type=jnp.float32)
    s = (s + b_ref[...]) * _rs(dego_ref[...])
    sa_ref[...] = s[:, :128]
    sb_ref[...] = s[:, 128:]


def _b3_body(oa_ref, ob_ref, degi_ref, w_ref, b_ref, out_ref):
    o = jnp.concatenate([oa_ref[...], ob_ref[...]], axis=1)
    h = jax.nn.relu(o * _rs(degi_ref[...]))
    out_ref[...] = jnp.dot(h, w_ref[...], preferred_element_type=jnp.float32) + b_ref[...]


def _blk_spec(rows, cols):
    return pl.BlockSpec((rows, cols), lambda i: (i, 0))


def _full_spec(r, c):
    return pl.BlockSpec((r, c), lambda i: (0, 0))


def _stage1(x, W0, b0, deg_out):
    return pl.pallas_call(
        _b1_body,
        grid=(NP // SROWS,),
        in_specs=[_blk_spec(SROWS, 128), _full_spec(128, 256),
                  _full_spec(1, 256), _blk_spec(SROWS, 1)],
        out_specs=[_blk_spec(SROWS, 128), _blk_spec(SROWS, 128)],
        out_shape=[jax.ShapeDtypeStruct((NP, 128), jnp.float32)] * 2,
    )(x, W0, b0.reshape(1, -1), deg_out)


def _stage2(oa, ob, deg_in, deg_out, W1, b1):
    return pl.pallas_call(
        _b2_body,
        grid=(NP // SROWS,),
        in_specs=[_blk_spec(SROWS, 128), _blk_spec(SROWS, 128),
                  _blk_spec(SROWS, 1), _blk_spec(SROWS, 1),
                  _full_spec(256, 256), _full_spec(1, 256)],
        out_specs=[_blk_spec(SROWS, 128), _blk_spec(SROWS, 128)],
        out_shape=[jax.ShapeDtypeStruct((NP, 128), jnp.float32)] * 2,
    )(oa, ob, deg_in, deg_out, W1, b1.reshape(1, -1))


def _stage3(oa, ob, deg_in, Wl, bl):
    nc = Wl.shape[1]
    return pl.pallas_call(
        _b3_body,
        grid=(N // ROWS,),
        in_specs=[_blk_spec(ROWS, 128), _blk_spec(ROWS, 128),
                  _blk_spec(ROWS, 1), _full_spec(256, nc), _full_spec(1, nc)],
        out_specs=_blk_spec(ROWS, nc),
        out_shape=jax.ShapeDtypeStruct((N, nc), jnp.float32),
    )(oa, ob, deg_in, Wl, bl.reshape(1, -1))


_MESH = dict(core_axis_name="c", subcore_axis_name="s")


def _tile_zero(s, zeros_hbm, acc_sh):
    sl = pl.ds(s * RPT, RPT)
    pltpu.sync_copy(zeros_hbm.at[sl], acc_sh.at[sl])


def _fill_ones(ones_v):
    for k in range(CH // 16):
        ones_v[pl.ds(k * 16, 16)] = jnp.ones((16,), jnp.float32)


def _deg_kernel(src2d, dst2d, zeros_hbm, dego_hbm, degi_hbm,
                idx_v, ones_v, acc_sh, ssem):
    c = lax.axis_index("c")
    s = lax.axis_index("s")
    _fill_ones(ones_v)

    def run(idx2d, out_hbm):
        _tile_zero(s, zeros_hbm, acc_sh)
        plsc.subcore_barrier()

        def block(bi, carry):
            cb = (s * DNBLK + bi) * DBLK
            pltpu.sync_copy(idx2d.at[pl.ds(cb, DBLK)], idx_v)
            descs = [
                pltpu.async_copy(ones_v, acc_sh.at[idx_v.at[j]], ssem,
                                 add=True)
                for j in range(DBLK)
            ]
            for d in descs:
                d.wait()
            return carry
        lax.fori_loop(0, DNBLK, block, 0)
        plsc.subcore_barrier()
        sl = pl.ds(s * RPT, RPT)
        pltpu.sync_copy(acc_sh.at[sl], out_hbm.at[sl])

    @pl.when(c == 0)
    def _():
        run(src2d, dego_hbm)

    @pl.when(c == 1)
    def _():
        run(dst2d, degi_hbm)


def _degrees(src2d, dst2d):
    zeros = jnp.zeros((NP,), jnp.float32)
    return pl.kernel(
        _deg_kernel,
        out_type=[jax.ShapeDtypeStruct((NP,), jnp.float32)] * 2,
        mesh=plsc.VectorSubcoreMesh(**_MESH),
        scratch_types=[
            pltpu.VMEM((DBLK, CH), jnp.int32),
            pltpu.VMEM((CH,), jnp.float32),
            pltpu.VMEM_SHARED((NP,), jnp.float32),
            pltpu.SemaphoreType.DMA,
        ],
    )(src2d, dst2d, zeros)


def _spmm_kernel(src2d, dst2d, sup_a, sup_b, zeros_hbm, oa_hbm, ob_hbm,
                 srcB, dstB, r0, r1, acc_sh, gsem, ssem):
    c = lax.axis_index("c")
    s = lax.axis_index("s")
    rows = (r0, r1)

    def run(sup_hbm, out_hbm):
        _tile_zero(s, zeros_hbm, acc_sh)
        plsc.subcore_barrier()

        def block(bi, carry):
            cb = (s * NBLK + bi) * BLK
            pltpu.sync_copy(src2d.at[pl.ds(cb, BLK)], srcB)
            pltpu.sync_copy(dst2d.at[pl.ds(cb, BLK)], dstB)
            scats = [None, None]
            g = pltpu.async_copy(sup_hbm.at[srcB.at[0]], rows[0], gsem)
            for j in range(BLK):
                g.wait()
                if j + 1 < BLK:
                    nb = (j + 1) % 2
                    if scats[nb] is not None:
                        scats[nb].wait()
                    g = pltpu.async_copy(sup_hbm.at[srcB.at[j + 1]],
                                         rows[nb], gsem)
                scats[j % 2] = pltpu.async_copy(
                    rows[j % 2], acc_sh.at[dstB.at[j]], ssem, add=True)
            for d in scats:
                d.wait()
            return carry
        lax.fori_loop(0, NBLK, block, 0)
        plsc.subcore_barrier()
        sl = pl.ds(s * RPT, RPT)
        pltpu.sync_copy(acc_sh.at[sl], out_hbm.at[sl])

    @pl.when(c == 0)
    def _():
        run(sup_a, oa_hbm)

    @pl.when(c == 1)
    def _():
        run(sup_b, ob_hbm)


def _spmm(src2d, dst2d, sup_a, sup_b):
    zeros = jnp.zeros((NP, 128), jnp.float32)
    return pl.kernel(
        _spmm_kernel,
        out_type=[jax.ShapeDtypeStruct((NP, 128), jnp.float32)] * 2,
        mesh=plsc.VectorSubcoreMesh(**_MESH),
        scratch_types=[
            pltpu.VMEM((BLK, CH), jnp.int32),
            pltpu.VMEM((BLK, CH), jnp.int32),
            pltpu.VMEM((CH, 128), jnp.float32),
            pltpu.VMEM((CH, 128), jnp.float32),
            pltpu.VMEM_SHARED((NP, 128), jnp.float32),
            pltpu.SemaphoreType.DMA,
            pltpu.SemaphoreType.DMA,
        ],
    )(src2d, dst2d, sup_a, sup_b, zeros)


def kernel(x, adj, W0, b0, W1, b1, Wl, bl):
    pad = EP - E
    pad_idx = N + (jnp.arange(pad, dtype=jnp.int32) % (NP - N))
    src2d = jnp.concatenate([adj[0], pad_idx]).reshape(NCH, CH)
    dst2d = jnp.concatenate([adj[1], pad_idx]).reshape(NCH, CH)

    deg_out, deg_in = _degrees(src2d, dst2d)
    dego = deg_out.reshape(NP, 1)
    degi = deg_in.reshape(NP, 1)
    sa, sb = _stage1(x, W0, b0, dego)

    limit = 2 - (adj[0, 0] >> 31)

    def cond(state):
        return state[0] < limit

    def body(state):
        i, sa_c, sb_c, _, _ = state
        oa, ob = _spmm(src2d, dst2d, sa_c, sb_c)
        sa_n, sb_n = _stage2(oa, ob, degi, dego, W1, b1)
        return (i + 1, sa_n, sb_n, oa, ob)

    z = jnp.zeros((NP, 128), jnp.float32)
    _, _, _, oa, ob = lax.while_loop(cond, body, (jnp.int32(0), sa, sb, z, z))
    return _stage3(oa, ob, degi, Wl, bl)


# speedup vs baseline: 1.0089x; 1.0089x over previous
import functools

import jax
import jax.numpy as jnp
from jax import lax
from jax.experimental import pallas as pl
from jax.experimental.pallas import tpu as pltpu
from jax.experimental.pallas import tpu_sc as plsc

N = 10000
E = 320000
NP = 10240
EP = 327680
CH = 128
NCH = EP // CH
NTILES = 16
CPT = NCH // NTILES
RPT = NP // NTILES

SROWS = 2560
ROWS = 2000

BLK = 8
NBLK = CPT // BLK
GS = (3, 3, 2)

DBLK = 16
DNBLK = CPT // DBLK


def _rs(d):
    return lax.rsqrt(jnp.where(d > 0, d, 1.0))


def _b1_body(x_ref, w_ref, b_ref, dego_ref, sa_ref, sb_ref):
    s = jnp.dot(x_ref[...], w_ref[...], preferred_element_type=jnp.float32)
    s = (s + b_ref[...]) * _rs(dego_ref[...])
    sa_ref[...] = s[:, :128]
    sb_ref[...] = s[:, 128:]


def _b2_body(oa_ref, ob_ref, degi_ref, dego_ref, w_ref, b_ref, sa_ref, sb_ref):
    o = jnp.concatenate([oa_ref[...], ob_ref[...]], axis=1)
    h = jax.nn.relu(o * _rs(degi_ref[...]))
    s = jnp.dot(h, w_ref[...], preferred_element_type=jnp.float32)
    s = (s + b_ref[...]) * _rs(dego_ref[...])
    sa_ref[...] = s[:, :128]
    sb_ref[...] = s[:, 128:]


def _b3_body(oa_ref, ob_ref, degi_ref, w_ref, b_ref, out_ref):
    o = jnp.concatenate([oa_ref[...], ob_ref[...]], axis=1)
    h = jax.nn.relu(o * _rs(degi_ref[...]))
    out_ref[...] = jnp.dot(h, w_ref[...], preferred_element_type=jnp.float32) + b_ref[...]


def _blk_spec(rows, cols):
    return pl.BlockSpec((rows, cols), lambda i: (i, 0))


def _full_spec(r, c):
    return pl.BlockSpec((r, c), lambda i: (0, 0))


def _stage1(x, W0, b0, deg_out):
    return pl.pallas_call(
        _b1_body,
        grid=(NP // SROWS,),
        in_specs=[_blk_spec(SROWS, 128), _full_spec(128, 256),
                  _full_spec(1, 256), _blk_spec(SROWS, 1)],
        out_specs=[_blk_spec(SROWS, 128), _blk_spec(SROWS, 128)],
        out_shape=[jax.ShapeDtypeStruct((NP, 128), jnp.float32)] * 2,
    )(x, W0, b0.reshape(1, -1), deg_out)


def _stage2(oa, ob, deg_in, deg_out, W1, b1):
    return pl.pallas_call(
        _b2_body,
        grid=(NP // SROWS,),
        in_specs=[_blk_spec(SROWS, 128), _blk_spec(SROWS, 128),
                  _blk_spec(SROWS, 1), _blk_spec(SROWS, 1),
                  _full_spec(256, 256), _full_spec(1, 256)],
        out_specs=[_blk_spec(SROWS, 128), _blk_spec(SROWS, 128)],
        out_shape=[jax.ShapeDtypeStruct((NP, 128), jnp.float32)] * 2,
    )(oa, ob, deg_in, deg_out, W1, b1.reshape(1, -1))


def _stage3(oa, ob, deg_in, Wl, bl):
    nc = Wl.shape[1]
    return pl.pallas_call(
        _b3_body,
        grid=(N // ROWS,),
        in_specs=[_blk_spec(ROWS, 128), _blk_spec(ROWS, 128),
                  _blk_spec(ROWS, 1), _full_spec(256, nc), _full_spec(1, nc)],
        out_specs=_blk_spec(ROWS, nc),
        out_shape=jax.ShapeDtypeStruct((N, nc), jnp.float32),
    )(oa, ob, deg_in, Wl, bl.reshape(1, -1))


_MESH = dict(core_axis_name="c", subcore_axis_name="s")


def _tile_zero(s, zeros_hbm, acc_sh):
    sl = pl.ds(s * RPT, RPT)
    pltpu.sync_copy(zeros_hbm.at[sl], acc_sh.at[sl])


def _fill_ones(ones_v):
    for k in range(CH // 16):
        ones_v[pl.ds(k * 16, 16)] = jnp.ones((16,), jnp.float32)


def _deg_kernel(src2d, dst2d, zeros_hbm, dego_hbm, degi_hbm,
                idx_v, ones_v, acc_sh, ssem):
    c = lax.axis_index("c")
    s = lax.axis_index("s")
    _fill_ones(ones_v)

    def run(idx2d, out_hbm):
        _tile_zero(s, zeros_hbm, acc_sh)
        plsc.subcore_barrier()

        def block(bi, carry):
            cb = (s * DNBLK + bi) * DBLK
            pltpu.sync_copy(idx2d.at[pl.ds(cb, DBLK)], idx_v)
            descs = [
                pltpu.async_copy(ones_v, acc_sh.at[idx_v.at[j]], ssem,
                                 add=True)
                for j in range(DBLK)
            ]
            for d in descs:
                d.wait()
            return carry
        lax.fori_loop(0, DNBLK, block, 0)
        plsc.subcore_barrier()
        sl = pl.ds(s * RPT, RPT)
        pltpu.sync_copy(acc_sh.at[sl], out_hbm.at[sl])

    @pl.when(c == 0)
    def _():
        run(src2d, dego_hbm)

    @pl.when(c == 1)
    def _():
        run(dst2d, degi_hbm)


def _degrees(src2d, dst2d):
    zeros = jnp.zeros((NP,), jnp.float32)
    return pl.kernel(
        _deg_kernel,
        out_type=[jax.ShapeDtypeStruct((NP,), jnp.float32)] * 2,
        mesh=plsc.VectorSubcoreMesh(**_MESH),
        scratch_types=[
            pltpu.VMEM((DBLK, CH), jnp.int32),
            pltpu.VMEM((CH,), jnp.float32),
            pltpu.VMEM_SHARED((NP,), jnp.float32),
            pltpu.SemaphoreType.DMA,
        ],
    )(src2d, dst2d, zeros)


def _spmm_kernel(src2d, dst2d, sup_a, sup_b, zeros_hbm, oa_hbm, ob_hbm,
                 srcB, dstB, r0, r1, acc_sh, gsem, ssem):
    c = lax.axis_index("c")
    s = lax.axis_index("s")
    rows = (r0, r1)

    def run(sup_hbm, out_hbm):
        _tile_zero(s, zeros_hbm, acc_sh)
        plsc.subcore_barrier()

        def block(bi, carry):
            cb = (s * NBLK + bi) * BLK
            pltpu.sync_copy(src2d.at[pl.ds(cb, BLK)], srcB)
            pltpu.sync_copy(dst2d.at[pl.ds(cb, BLK)], dstB)
            scats = [None, None]
            g = pltpu.async_copy(sup_hbm.at[srcB.at[0]], rows[0], gsem)
            for j in range(BLK):
                g.wait()
                if j + 1 < BLK:
                    nb = (j + 1) % 2
                    if scats[nb] is not None:
                        scats[nb].wait()
                    g = pltpu.async_copy(sup_hbm.at[srcB.at[j + 1]],
                                         rows[nb], gsem)
                scats[j % 2] = pltpu.async_copy(
                    rows[j % 2], acc_sh.at[dstB.at[j]], ssem, add=True)
            for d in scats:
                d.wait()
            return carry
        lax.fori_loop(0, NBLK, block, 0)
        plsc.subcore_barrier()
        sl = pl.ds(s * RPT, RPT)
        pltpu.sync_copy(acc_sh.at[sl], out_hbm.at[sl])

    @pl.when(c == 0)
    def _():
        run(sup_a, oa_hbm)

    @pl.when(c == 1)
    def _():
        run(sup_b, ob_hbm)


def _spmm(src2d, dst2d, sup_a, sup_b):
    zeros = jnp.zeros((NP, 128), jnp.float32)
    return pl.kernel(
        _spmm_kernel,
        out_type=[jax.ShapeDtypeStruct((NP, 128), jnp.float32)] * 2,
        mesh=plsc.VectorSubcoreMesh(**_MESH),
        scratch_types=[
            pltpu.VMEM((BLK, CH), jnp.int32),
            pltpu.VMEM((BLK, CH), jnp.int32),
            pltpu.VMEM((CH, 128), jnp.float32),
            pltpu.VMEM((CH, 128), jnp.float32),
            pltpu.VMEM_SHARED((NP, 128), jnp.float32),
            pltpu.SemaphoreType.DMA,
            pltpu.SemaphoreType.DMA,
        ],
    )(src2d, dst2d, sup_a, sup_b, zeros)


def kernel(x, adj, W0, b0, W1, b1, Wl, bl):
    pad = EP - E
    pad_idx = N + (jnp.arange(pad, dtype=jnp.int32) % (NP - N))
    src2d = jnp.concatenate([adj[0], pad_idx]).reshape(NCH, CH)
    dst2d = jnp.concatenate([adj[1], pad_idx]).reshape(NCH, CH)

    deg_out, deg_in = _degrees(src2d, dst2d)
    dego = deg_out.reshape(NP, 1)
    degi = deg_in.reshape(NP, 1)
    sa, sb = _stage1(x, W0, b0, dego)

    limit = 2 - (adj[0, 0] >> 31)

    def cond(state):
        return state[0] < limit

    def body(state):
        i, sa_c, sb_c, _, _ = state
        oa, ob = _spmm(src2d, dst2d, sa_c, sb_c)
        sa_n, sb_n = lax.cond(
            i < 1,
            lambda: _stage2(oa, ob, degi, dego, W1, b1),
            lambda: (sa_c, sb_c))
        return (i + 1, sa_n, sb_n, oa, ob)

    z = jnp.zeros((NP, 128), jnp.float32)
    _, _, _, oa, ob = lax.while_loop(cond, body, (jnp.int32(0), sa, sb, z, z))
    return _stage3(oa, ob, degi, Wl, bl)


# speedup vs baseline: 1.0783x; 1.0687x over previous
import functools

import jax
import jax.numpy as jnp
from jax import lax
from jax.experimental import pallas as pl
from jax.experimental.pallas import tpu as pltpu
from jax.experimental.pallas import tpu_sc as plsc

N = 10000
E = 320000
NP = 10240
EP = 327680
CH = 128
NCH = EP // CH
NTILES = 16
CPT = NCH // NTILES
RPT = NP // NTILES

SROWS = 2560
ROWS = 2000

IBLK = 32
NIBLK = CPT // IBLK
UNR = 8

DBLK = 16
DNBLK = CPT // DBLK


def _rs(d):
    return lax.rsqrt(jnp.where(d > 0, d, 1.0))


def _b1_body(x_ref, w_ref, b_ref, dego_ref, sa_ref, sb_ref):
    s = jnp.dot(x_ref[...], w_ref[...], preferred_element_type=jnp.float32)
    s = (s + b_ref[...]) * _rs(dego_ref[...])
    sa_ref[...] = s[:, :128]
    sb_ref[...] = s[:, 128:]


def _b2_body(oa_ref, ob_ref, degi_ref, dego_ref, w_ref, b_ref, sa_ref, sb_ref):
    o = jnp.concatenate([oa_ref[...], ob_ref[...]], axis=1)
    h = jax.nn.relu(o * _rs(degi_ref[...]))
    s = jnp.dot(h, w_ref[...], preferred_element_type=jnp.float32)
    s = (s + b_ref[...]) * _rs(dego_ref[...])
    sa_ref[...] = s[:, :128]
    sb_ref[...] = s[:, 128:]


def _b3_body(oa_ref, ob_ref, degi_ref, w_ref, b_ref, out_ref):
    o = jnp.concatenate([oa_ref[...], ob_ref[...]], axis=1)
    h = jax.nn.relu(o * _rs(degi_ref[...]))
    out_ref[...] = jnp.dot(h, w_ref[...], preferred_element_type=jnp.float32) + b_ref[...]


def _blk_spec(rows, cols):
    return pl.BlockSpec((rows, cols), lambda i: (i, 0))


def _full_spec(r, c):
    return pl.BlockSpec((r, c), lambda i: (0, 0))


def _stage1(x, W0, b0, deg_out):
    return pl.pallas_call(
        _b1_body,
        grid=(NP // SROWS,),
        in_specs=[_blk_spec(SROWS, 128), _full_spec(128, 256),
                  _full_spec(1, 256), _blk_spec(SROWS, 1)],
        out_specs=[_blk_spec(SROWS, 128), _blk_spec(SROWS, 128)],
        out_shape=[jax.ShapeDtypeStruct((NP, 128), jnp.float32)] * 2,
    )(x, W0, b0.reshape(1, -1), deg_out)


def _stage2(oa, ob, deg_in, deg_out, W1, b1):
    return pl.pallas_call(
        _b2_body,
        grid=(NP // SROWS,),
        in_specs=[_blk_spec(SROWS, 128), _blk_spec(SROWS, 128),
                  _blk_spec(SROWS, 1), _blk_spec(SROWS, 1),
                  _full_spec(256, 256), _full_spec(1, 256)],
        out_specs=[_blk_spec(SROWS, 128), _blk_spec(SROWS, 128)],
        out_shape=[jax.ShapeDtypeStruct((NP, 128), jnp.float32)] * 2,
    )(oa, ob, deg_in, deg_out, W1, b1.reshape(1, -1))


def _stage3(oa, ob, deg_in, Wl, bl):
    nc = Wl.shape[1]
    return pl.pallas_call(
        _b3_body,
        grid=(N // ROWS,),
        in_specs=[_blk_spec(ROWS, 128), _blk_spec(ROWS, 128),
                  _blk_spec(ROWS, 1), _full_spec(256, nc), _full_spec(1, nc)],
        out_specs=_blk_spec(ROWS, nc),
        out_shape=jax.ShapeDtypeStruct((N, nc), jnp.float32),
    )(oa, ob, deg_in, Wl, bl.reshape(1, -1))


_MESH = dict(core_axis_name="c", subcore_axis_name="s")


def _tile_zero(s, zeros_hbm, acc_sh):
    sl = pl.ds(s * RPT, RPT)
    pltpu.sync_copy(zeros_hbm.at[sl], acc_sh.at[sl])


def _fill_ones(ones_v):
    for k in range(CH // 16):
        ones_v[pl.ds(k * 16, 16)] = jnp.ones((16,), jnp.float32)


def _deg_kernel(src2d, dst2d, zeros_hbm, dego_hbm, degi_hbm,
                idx_v, ones_v, acc_sh, ssem):
    c = lax.axis_index("c")
    s = lax.axis_index("s")
    _fill_ones(ones_v)

    def run(idx2d, out_hbm):
        _tile_zero(s, zeros_hbm, acc_sh)
        plsc.subcore_barrier()

        def block(bi, carry):
            cb = (s * DNBLK + bi) * DBLK
            pltpu.sync_copy(idx2d.at[pl.ds(cb, DBLK)], idx_v)
            descs = [
                pltpu.async_copy(ones_v, acc_sh.at[idx_v.at[j]], ssem,
                                 add=True)
                for j in range(DBLK)
            ]
            for d in descs:
                d.wait()
            return carry
        lax.fori_loop(0, DNBLK, block, 0)
        plsc.subcore_barrier()
        sl = pl.ds(s * RPT, RPT)
        pltpu.sync_copy(acc_sh.at[sl], out_hbm.at[sl])

    @pl.when(c == 0)
    def _():
        run(src2d, dego_hbm)

    @pl.when(c == 1)
    def _():
        run(dst2d, degi_hbm)


def _degrees(src2d, dst2d):
    zeros = jnp.zeros((NP,), jnp.float32)
    return pl.kernel(
        _deg_kernel,
        out_type=[jax.ShapeDtypeStruct((NP,), jnp.float32)] * 2,
        mesh=plsc.VectorSubcoreMesh(**_MESH),
        scratch_types=[
            pltpu.VMEM((DBLK, CH), jnp.int32),
            pltpu.VMEM((CH,), jnp.float32),
            pltpu.VMEM_SHARED((NP,), jnp.float32),
            pltpu.SemaphoreType.DMA,
        ],
    )(src2d, dst2d, zeros)


def _spmm_kernel(src2d, dst2d, sup_a, sup_b, zeros_hbm, oa_hbm, ob_hbm,
                 srcB, dstB, r0, r1, acc_sh, gsem, ssem):
    c = lax.axis_index("c")
    s = lax.axis_index("s")
    rows = (r0, r1)

    def run(sup_hbm, out_hbm):
        _tile_zero(s, zeros_hbm, acc_sh)
        plsc.subcore_barrier()

        def block(bi, carry):
            cb = (s * NIBLK + bi) * IBLK
            ia = pltpu.async_copy(src2d.at[pl.ds(cb, IBLK)], srcB, gsem)
            ib = pltpu.async_copy(dst2d.at[pl.ds(cb, IBLK)], dstB, gsem)
            ia.wait()
            ib.wait()

            def step(ji, c2):
                j0 = ji * UNR
                scats = [None, None]
                g = pltpu.async_copy(sup_hbm.at[srcB.at[j0]], rows[0], gsem)
                for j in range(UNR):
                    g.wait()
                    if j + 1 < UNR:
                        nb = (j + 1) % 2
                        if scats[nb] is not None:
                            scats[nb].wait()
                        g = pltpu.async_copy(sup_hbm.at[srcB.at[j0 + j + 1]],
                                             rows[nb], gsem)
                    scats[j % 2] = pltpu.async_copy(
                        rows[j % 2], acc_sh.at[dstB.at[j0 + j]], ssem,
                        add=True)
                for d in scats:
                    d.wait()
                return c2
            lax.fori_loop(0, IBLK // UNR, step, 0)
            return carry
        lax.fori_loop(0, NIBLK, block, 0)
        plsc.subcore_barrier()
        sl = pl.ds(s * RPT, RPT)
        pltpu.sync_copy(acc_sh.at[sl], out_hbm.at[sl])

    @pl.when(c == 0)
    def _():
        run(sup_a, oa_hbm)

    @pl.when(c == 1)
    def _():
        run(sup_b, ob_hbm)


def _spmm(src2d, dst2d, sup_a, sup_b):
    zeros = jnp.zeros((NP, 128), jnp.float32)
    return pl.kernel(
        _spmm_kernel,
        out_type=[jax.ShapeDtypeStruct((NP, 128), jnp.float32)] * 2,
        mesh=plsc.VectorSubcoreMesh(**_MESH),
        scratch_types=[
            pltpu.VMEM((IBLK, CH), jnp.int32),
            pltpu.VMEM((IBLK, CH), jnp.int32),
            pltpu.VMEM((CH, 128), jnp.float32),
            pltpu.VMEM((CH, 128), jnp.float32),
            pltpu.VMEM_SHARED((NP, 128), jnp.float32),
            pltpu.SemaphoreType.DMA,
            pltpu.SemaphoreType.DMA,
        ],
    )(src2d, dst2d, sup_a, sup_b, zeros)


def kernel(x, adj, W0, b0, W1, b1, Wl, bl):
    pad = EP - E
    pad_idx = N + (jnp.arange(pad, dtype=jnp.int32) % (NP - N))
    src2d = jnp.concatenate([adj[0], pad_idx]).reshape(NCH, CH)
    dst2d = jnp.concatenate([adj[1], pad_idx]).reshape(NCH, CH)

    deg_out, deg_in = _degrees(src2d, dst2d)
    dego = deg_out.reshape(NP, 1)
    degi = deg_in.reshape(NP, 1)
    sa, sb = _stage1(x, W0, b0, dego)

    limit = 2 - (adj[0, 0] >> 31)

    def cond(state):
        return state[0] < limit

    def body(state):
        i, sa_c, sb_c, _, _ = state
        oa, ob = _spmm(src2d, dst2d, sa_c, sb_c)
        sa_n, sb_n = lax.cond(
            i < 1,
            lambda: _stage2(oa, ob, degi, dego, W1, b1),
            lambda: (sa_c, sb_c))
        return (i + 1, sa_n, sb_n, oa, ob)

    z = jnp.zeros((NP, 128), jnp.float32)
    _, _, _, oa, ob = lax.while_loop(cond, body, (jnp.int32(0), sa, sb, z, z))
    return _stage3(oa, ob, degi, Wl, bl)
